# baseline (speedup 1.0000x reference)
import functools

import jax
import jax.numpy as jnp
from jax import lax
from jax.experimental import pallas as pl
from jax.experimental.pallas import tpu as pltpu
from jax.experimental.pallas import tpu_sc as plsc

_LANES = 16


@functools.lru_cache(maxsize=None)
def _build_sc_kernel(B, H, W, C):
    ROWS = 2 * C
    HW = H * W
    NC, NS = 2, 16
    NW = NC * NS
    RPW = ROWS // NW
    CHUNK = RPW * HW
    NITER = CHUNK // _LANES

    assert ROWS % NW == 0 and CHUNK % _LANES == 0 and CHUNK % 8 == 0

    mesh = plsc.VectorSubcoreMesh(core_axis_name="c", subcore_axis_name="s")

    @functools.partial(
        pl.kernel,
        mesh=mesh,
        out_type=jax.ShapeDtypeStruct((B * ROWS * HW,), jnp.float32),
        scratch_types=[
            pltpu.VMEM((H * ROWS,), jnp.float32),
            pltpu.VMEM((CHUNK,), jnp.float32),
            pltpu.SemaphoreType.DMA,
        ],
        compiler_params=pltpu.CompilerParams(needs_layout_passes=False),
    )
    def sc_kernel(emb_hbm, out_hbm, emb_v, buf_v, sem):
        wid = lax.axis_index("s") * NC + lax.axis_index("c")
        c0 = wid * RPW

        pltpu.sync_copy(emb_hbm, emb_v)

        lane = lax.iota(jnp.int32, _LANES)
        w0 = lane % W
        h0 = (lane // W) % H
        r0 = lane // HW

        def body(p, carry):
            w, h, r = carry
            c = r + c0
            spat = jnp.where(c < C, w, h)
            val = plsc.load_gather(emb_v, [spat * ROWS + c])
            buf_v[pl.ds(p * _LANES, _LANES)] = val
            w = w + _LANES
            ovf_w = w >= W
            w = jnp.where(ovf_w, w - W, w)
            h = jnp.where(ovf_w, h + 1, h)
            ovf_h = h >= H
            h = jnp.where(ovf_h, h - H, h)
            r = jnp.where(ovf_h, r + 1, r)
            return w, h, r

        lax.fori_loop(0, NITER, body, (w0, h0, r0))

        copies = []
        for b in range(B):
            off = b * ROWS * HW + c0 * HW
            copies.append(
                pltpu.make_async_copy(buf_v, out_hbm.at[pl.ds(off, CHUNK)], sem)
            )
        for cp in copies:
            cp.start()
        for cp in copies:
            cp.wait()

    return sc_kernel


def kernel(mask, row_embed, col_embed):
    B, H, W = mask.shape
    C = col_embed.shape[1]
    emb = jnp.concatenate([col_embed, row_embed], axis=1).reshape(-1)
    out_flat = _build_sc_kernel(B, H, W, C)(emb)
    return out_flat.reshape(B, 2 * C, H, W)


# speedup vs baseline: 1.8606x; 1.8606x over previous
import functools

import jax
import jax.numpy as jnp
from jax.experimental import pallas as pl
from jax.experimental.pallas import tpu as pltpu


@functools.lru_cache(maxsize=None)
def _build_tc_kernel(B, H, W, C, CB):
    NBLK = 2 * C // CB
    NCB = C // CB

    def body(colT_ref, rowT_ref, out_ref):
        i = pl.program_id(1)

        @pl.when(i < NCB)
        def _():
            out_ref[...] = jnp.broadcast_to(
                colT_ref[...][None, :, None, :], (1, CB, H, W)
            )

        @pl.when(i >= NCB)
        def _():
            out_ref[...] = jnp.broadcast_to(
                rowT_ref[...][None, :, :, None], (1, CB, H, W)
            )

    grid = (B, NBLK)
    return pl.pallas_call(
        body,
        grid=grid,
        in_specs=[
            pl.BlockSpec((CB, W), lambda b, i: (i % NCB, 0)),
            pl.BlockSpec((CB, H), lambda b, i: (i % NCB, 0)),
        ],
        out_specs=pl.BlockSpec((1, CB, H, W), lambda b, i: (b, i, 0, 0)),
        out_shape=jax.ShapeDtypeStruct((B, 2 * C, H, W), jnp.float32),
        compiler_params=pltpu.CompilerParams(
            dimension_semantics=("parallel", "arbitrary"),
        ),
    )


def kernel(mask, row_embed, col_embed):
    B, H, W = mask.shape
    C = col_embed.shape[1]
    colT = col_embed.T
    rowT = row_embed.T
    return _build_tc_kernel(B, H, W, C, 64)(colT, rowT)
